# baseline (speedup 1.0000x reference)
import functools

import jax
import jax.numpy as jnp
from jax import lax
from jax.experimental import pallas as pl
from jax.experimental.pallas import tpu as pltpu
from jax.experimental.pallas import tpu_sc as plsc

N = 10000
DEG = 32
D = 128

_SLICES = (400, 2400, 2400, 2400, 2400)


_SUP = 256


def _make_sc_gather(n_nodes):
    rows_total = n_nodes * DEG
    nsup_tot = rows_total // _SUP
    info = plsc.get_sparse_core_info()
    nw = info.num_cores * info.num_subcores
    nfull = nsup_tot // nw
    extra = nsup_tot % nw
    maxi = (nfull + (1 if extra else 0)) * _SUP
    nit = (nfull + (1 if extra else 0) + 1) // 2

    mesh = plsc.VectorSubcoreMesh(core_axis_name="c", subcore_axis_name="s")

    @functools.partial(
        pl.kernel,
        mesh=mesh,
        out_type=[jax.ShapeDtypeStruct((rows_total, D), jnp.float32)] * 2,
        scratch_types=[
            pltpu.VMEM((2, maxi), jnp.int32),
            pltpu.VMEM((_SUP, D), jnp.float32),
            pltpu.VMEM((_SUP, D), jnp.float32),
            pltpu.SemaphoreType.DMA,
            pltpu.SemaphoreType.DMA,
            pltpu.SemaphoreType.DMA,
            pltpu.SemaphoreType.DMA,
        ],
    )
    def gather(x_hbm, i0_hbm, i1_hbm, o0_hbm, o1_hbm,
               idx_v, rows0, rows1, gsem0, gsem1, osem0, osem1):
        wid = lax.axis_index("s") * info.num_cores + lax.axis_index("c")
        nsup = nfull + jnp.where(wid < extra, 1, 0)
        base_row = (wid * nfull + jnp.minimum(wid, extra)) * _SUP
        rows = (rows0, rows1)
        gsem = (gsem0, gsem1)
        osem = (osem0, osem1)

        pltpu.sync_copy(i0_hbm.at[pl.ds(base_row, maxi)], idx_v.at[0])
        pltpu.sync_copy(i1_hbm.at[pl.ds(base_row, maxi)], idx_v.at[1])

        def start_gathers(r, i, b):
            for k in range(_SUP // 128):
                pltpu.async_copy(
                    x_hbm.at[idx_v.at[r, pl.ds(i * _SUP + k * 128, 128)]],
                    rows[b].at[pl.ds(k * 128, 128)],
                    gsem[b])

        def wait_gathers(b):
            pltpu.make_async_copy(x_hbm.at[pl.ds(0, _SUP)], rows[b],
                                  gsem[b]).wait()

        def start_out(out_hbm, i, b):
            pltpu.async_copy(rows[b],
                             out_hbm.at[pl.ds(base_row + i * _SUP, _SUP)],
                             osem[b])

        def wait_out(out_hbm, b):
            pltpu.make_async_copy(rows[b], out_hbm.at[pl.ds(0, _SUP)],
                                  osem[b]).wait()

        def one_rel(r, out_hbm):
            def body(j, carry):
                for b in (0, 1):
                    i = 2 * j + b

                    @pl.when(i < nsup)
                    def _(i=i, b=b):
                        @pl.when(i >= 2)
                        def _():
                            wait_out(out_hbm, b)

                        start_gathers(r, i, b)

                        @pl.when(i >= 1)
                        def _():
                            wait_gathers(1 - b)
                            start_out(out_hbm, i - 1, 1 - b)

                return carry

            lax.fori_loop(0, nit, body, 0)
            for b in (0, 1):
                @pl.when((nsup - 1) % 2 == b)
                def _(b=b):
                    wait_gathers(b)
                    start_out(out_hbm, nsup - 1, b)
            wait_out(out_hbm, 0)

            @pl.when(nsup >= 2)
            def _():
                wait_out(out_hbm, 1)

        one_rel(0, o0_hbm)
        one_rel(1, o1_hbm)

    return gather


_BLK = 400
_UNROLL = 8


def _tc_body(x_ref, f0_ref, f1_ref,
             wcat0_ref, bc0_ref, ws0_ref, bs0_ref, wn0_ref,
             wcat1_ref, bc1_ref, ws1_ref, bs1_ref, wn1_ref,
             trans_ref, out_ref):
    xb = x_ref[...]
    w0 = wcat0_ref[...]
    w1 = wcat1_ref[...]
    b0 = bc0_ref[...]
    b1 = bc1_ref[...]

    def upd(g, c):
        i_ = 0.5 * jnp.tanh(g[:, 0:D]) + 0.5
        f_ = 0.5 * jnp.tanh(g[:, D:2 * D]) + 0.5
        gg = jnp.tanh(g[:, 2 * D:3 * D])
        o_ = 0.5 * jnp.tanh(g[:, 3 * D:4 * D]) + 0.5
        c2 = f_ * c + i_ * gg
        return o_ * jnp.tanh(c2), c2

    def step(j, carry):
        for k in range(_UNROLL):
            t = _UNROLL * j + k
            h0, c0, h1, c1 = carry
            xh0 = jnp.concatenate([f0_ref[t].astype(jnp.bfloat16),
                                   h0.astype(jnp.bfloat16)], axis=1)
            xh1 = jnp.concatenate([f1_ref[t].astype(jnp.bfloat16),
                                   h1.astype(jnp.bfloat16)], axis=1)
            g0 = jnp.dot(xh0, w0, preferred_element_type=jnp.float32) + b0
            g1 = jnp.dot(xh1, w1, preferred_element_type=jnp.float32) + b1
            h0n, c0n = upd(g0, c0)
            h1n, c1n = upd(g1, c1)
            carry = (h0n, c0n, h1n, c1n)
        return carry

    z = jnp.zeros((_BLK, D), jnp.float32)
    h0, _, h1, _ = lax.fori_loop(0, DEG // _UNROLL, step, (z, z, z, z))

    xbb = xb.astype(jnp.bfloat16)

    def proj(h, ws_ref, bs_ref, wn_ref):
        r = (jnp.dot(xbb, ws_ref[...], preferred_element_type=jnp.float32)
             + bs_ref[...]
             + jnp.dot(h.astype(jnp.bfloat16), wn_ref[...],
                       preferred_element_type=jnp.float32))
        return jnp.where(r > 0, r, 0.01 * r)

    h0 = proj(h0, ws0_ref, bs0_ref, wn0_ref)
    h1 = proj(h1, ws1_ref, bs1_ref, wn1_ref)

    tr = trans_ref[...]
    a0 = jnp.sum(jnp.dot(h0.astype(jnp.bfloat16), tr,
                         preferred_element_type=jnp.float32) * xb,
                 axis=1, keepdims=True)
    a1 = jnp.sum(jnp.dot(h1.astype(jnp.bfloat16), tr,
                         preferred_element_type=jnp.float32) * xb,
                 axis=1, keepdims=True)
    m = jnp.maximum(a0, a1)
    e0 = jnp.exp(a0 - m)
    e1 = jnp.exp(a1 - m)
    out_ref[...] = (e0 * h0 + e1 * h1) / (e0 + e1)


def _full(shape):
    return pl.BlockSpec(shape, lambda i: tuple(0 for _ in shape))


def _tc_call(x_slice, f0, f1, weights):
    n_sl = x_slice.shape[0]
    grid = n_sl // _BLK
    assert n_sl % _BLK == 0
    return pl.pallas_call(
        _tc_body,
        grid=(grid,),
        in_specs=[
            pl.BlockSpec((_BLK, D), lambda i: (i, 0)),
            pl.BlockSpec((DEG, _BLK, D), lambda i: (0, i, 0)),
            pl.BlockSpec((DEG, _BLK, D), lambda i: (0, i, 0)),
            _full((2 * D, 4 * D)), _full((1, 4 * D)),
            _full((D, D)), _full((1, D)), _full((D, D)),
            _full((2 * D, 4 * D)), _full((1, 4 * D)),
            _full((D, D)), _full((1, D)), _full((D, D)),
            _full((D, D)),
        ],
        out_specs=pl.BlockSpec((_BLK, D), lambda i: (i, 0)),
        out_shape=jax.ShapeDtypeStruct((n_sl, D), jnp.float32),
    )(x_slice, f0, f1, *weights)


def kernel(x, nbr_0, nbr_1, trans,
           W_ih_0, W_hh_0, b_ih_0, b_hh_0, W_self_0, b_self_0, W_neigh_0,
           W_ih_1, W_hh_1, b_ih_1, b_hh_1, W_self_1, b_self_1, W_neigh_1):
    bf = jnp.bfloat16
    gate_scale = jnp.concatenate(
        [jnp.full((2 * D,), 0.5, jnp.float32),
         jnp.ones((D,), jnp.float32),
         jnp.full((D,), 0.5, jnp.float32)])
    wcat0 = (jnp.concatenate([W_ih_0, W_hh_0], axis=1).T
             * gate_scale).astype(bf)
    wcat1 = (jnp.concatenate([W_ih_1, W_hh_1], axis=1).T
             * gate_scale).astype(bf)
    bc0 = ((b_ih_0 + b_hh_0) * gate_scale).reshape(1, 4 * D)
    bc1 = ((b_ih_1 + b_hh_1) * gate_scale).reshape(1, 4 * D)
    weights = (wcat0, bc0, W_self_0.T.astype(bf), b_self_0.reshape(1, D),
               W_neigh_0.T.astype(bf),
               wcat1, bc1, W_self_1.T.astype(bf), b_self_1.reshape(1, D),
               W_neigh_1.T.astype(bf),
               trans.astype(bf))

    gathers = {}
    pad = jnp.zeros((_SUP,), jnp.int32)
    outs = []
    base = 0
    for n_sl in _SLICES:
        if n_sl not in gathers:
            gathers[n_sl] = _make_sc_gather(n_sl)
        sl = slice(base, base + n_sl)
        base += n_sl
        idx0 = jnp.concatenate([nbr_0[sl].astype(jnp.int32).T.reshape(-1),
                                pad])
        idx1 = jnp.concatenate([nbr_1[sl].astype(jnp.int32).T.reshape(-1),
                                pad])
        f0_flat, f1_flat = gathers[n_sl](x, idx0, idx1)
        f0 = f0_flat.reshape(DEG, n_sl, D)
        f1 = f1_flat.reshape(DEG, n_sl, D)
        outs.append(_tc_call(x[sl], f0, f1, weights))
    return jnp.concatenate(outs, axis=0)


# speedup vs baseline: 1.0341x; 1.0341x over previous
import functools

import jax
import jax.numpy as jnp
from jax import lax
from jax.experimental import pallas as pl
from jax.experimental.pallas import tpu as pltpu
from jax.experimental.pallas import tpu_sc as plsc

N = 10000
DEG = 32
D = 128

_SLICES = (400, 800, 1600, 2400, 2400, 2400)


_SUP = 256


def _make_sc_gather(n_nodes):
    rows_total = n_nodes * DEG
    nsup_tot = rows_total // _SUP
    info = plsc.get_sparse_core_info()
    nw = info.num_cores * info.num_subcores
    nfull = nsup_tot // nw
    extra = nsup_tot % nw
    maxi = (nfull + (1 if extra else 0)) * _SUP
    nit = (nfull + (1 if extra else 0) + 1) // 2

    mesh = plsc.VectorSubcoreMesh(core_axis_name="c", subcore_axis_name="s")

    @functools.partial(
        pl.kernel,
        mesh=mesh,
        out_type=[jax.ShapeDtypeStruct((rows_total, D), jnp.float32)] * 2,
        scratch_types=[
            pltpu.VMEM((2, maxi), jnp.int32),
            pltpu.VMEM((_SUP, D), jnp.float32),
            pltpu.VMEM((_SUP, D), jnp.float32),
            pltpu.SemaphoreType.DMA,
            pltpu.SemaphoreType.DMA,
            pltpu.SemaphoreType.DMA,
            pltpu.SemaphoreType.DMA,
        ],
    )
    def gather(x_hbm, i0_hbm, i1_hbm, o0_hbm, o1_hbm,
               idx_v, rows0, rows1, gsem0, gsem1, osem0, osem1):
        wid = lax.axis_index("s") * info.num_cores + lax.axis_index("c")
        nsup = nfull + jnp.where(wid < extra, 1, 0)
        base_row = (wid * nfull + jnp.minimum(wid, extra)) * _SUP
        rows = (rows0, rows1)
        gsem = (gsem0, gsem1)
        osem = (osem0, osem1)

        pltpu.sync_copy(i0_hbm.at[pl.ds(base_row, maxi)], idx_v.at[0])
        pltpu.sync_copy(i1_hbm.at[pl.ds(base_row, maxi)], idx_v.at[1])

        def start_gathers(r, i, b):
            for k in range(_SUP // 128):
                pltpu.async_copy(
                    x_hbm.at[idx_v.at[r, pl.ds(i * _SUP + k * 128, 128)]],
                    rows[b].at[pl.ds(k * 128, 128)],
                    gsem[b])

        def wait_gathers(b):
            pltpu.make_async_copy(x_hbm.at[pl.ds(0, _SUP)], rows[b],
                                  gsem[b]).wait()

        def start_out(out_hbm, i, b):
            pltpu.async_copy(rows[b],
                             out_hbm.at[pl.ds(base_row + i * _SUP, _SUP)],
                             osem[b])

        def wait_out(out_hbm, b):
            pltpu.make_async_copy(rows[b], out_hbm.at[pl.ds(0, _SUP)],
                                  osem[b]).wait()

        def one_rel(r, out_hbm):
            def body(j, carry):
                for b in (0, 1):
                    i = 2 * j + b

                    @pl.when(i < nsup)
                    def _(i=i, b=b):
                        @pl.when(i >= 2)
                        def _():
                            wait_out(out_hbm, b)

                        start_gathers(r, i, b)

                        @pl.when(i >= 1)
                        def _():
                            wait_gathers(1 - b)
                            start_out(out_hbm, i - 1, 1 - b)

                return carry

            lax.fori_loop(0, nit, body, 0)
            for b in (0, 1):
                @pl.when((nsup - 1) % 2 == b)
                def _(b=b):
                    wait_gathers(b)
                    start_out(out_hbm, nsup - 1, b)
            wait_out(out_hbm, 0)

            @pl.when(nsup >= 2)
            def _():
                wait_out(out_hbm, 1)

        one_rel(0, o0_hbm)
        one_rel(1, o1_hbm)

    return gather


_BLK = 400
_UNROLL = 8


def _tc_body(x_ref, f0_ref, f1_ref,
             wcat0_ref, bc0_ref, ws0_ref, bs0_ref, wn0_ref,
             wcat1_ref, bc1_ref, ws1_ref, bs1_ref, wn1_ref,
             trans_ref, out_ref):
    xb = x_ref[...]
    w0 = wcat0_ref[...]
    w1 = wcat1_ref[...]
    b0 = bc0_ref[...]
    b1 = bc1_ref[...]

    def upd(g, c):
        i_ = 0.5 * jnp.tanh(g[:, 0:D]) + 0.5
        f_ = 0.5 * jnp.tanh(g[:, D:2 * D]) + 0.5
        gg = jnp.tanh(g[:, 2 * D:3 * D])
        o_ = 0.5 * jnp.tanh(g[:, 3 * D:4 * D]) + 0.5
        c2 = f_ * c + i_ * gg
        return o_ * jnp.tanh(c2), c2

    def step(j, carry):
        for k in range(_UNROLL):
            t = _UNROLL * j + k
            h0, c0, h1, c1 = carry
            xh0 = jnp.concatenate([f0_ref[t].astype(jnp.bfloat16),
                                   h0.astype(jnp.bfloat16)], axis=1)
            xh1 = jnp.concatenate([f1_ref[t].astype(jnp.bfloat16),
                                   h1.astype(jnp.bfloat16)], axis=1)
            g0 = jnp.dot(xh0, w0, preferred_element_type=jnp.float32) + b0
            g1 = jnp.dot(xh1, w1, preferred_element_type=jnp.float32) + b1
            h0n, c0n = upd(g0, c0)
            h1n, c1n = upd(g1, c1)
            carry = (h0n, c0n, h1n, c1n)
        return carry

    z = jnp.zeros((_BLK, D), jnp.float32)
    h0, _, h1, _ = lax.fori_loop(0, DEG // _UNROLL, step, (z, z, z, z))

    xbb = xb.astype(jnp.bfloat16)

    def proj(h, ws_ref, bs_ref, wn_ref):
        r = (jnp.dot(xbb, ws_ref[...], preferred_element_type=jnp.float32)
             + bs_ref[...]
             + jnp.dot(h.astype(jnp.bfloat16), wn_ref[...],
                       preferred_element_type=jnp.float32))
        return jnp.where(r > 0, r, 0.01 * r)

    h0 = proj(h0, ws0_ref, bs0_ref, wn0_ref)
    h1 = proj(h1, ws1_ref, bs1_ref, wn1_ref)

    tr = trans_ref[...]
    a0 = jnp.sum(jnp.dot(h0.astype(jnp.bfloat16), tr,
                         preferred_element_type=jnp.float32) * xb,
                 axis=1, keepdims=True)
    a1 = jnp.sum(jnp.dot(h1.astype(jnp.bfloat16), tr,
                         preferred_element_type=jnp.float32) * xb,
                 axis=1, keepdims=True)
    m = jnp.maximum(a0, a1)
    e0 = jnp.exp(a0 - m)
    e1 = jnp.exp(a1 - m)
    out_ref[...] = (e0 * h0 + e1 * h1) / (e0 + e1)


def _full(shape):
    return pl.BlockSpec(shape, lambda i: tuple(0 for _ in shape))


def _tc_call(x_slice, f0, f1, weights):
    n_sl = x_slice.shape[0]
    grid = n_sl // _BLK
    assert n_sl % _BLK == 0
    return pl.pallas_call(
        _tc_body,
        grid=(grid,),
        in_specs=[
            pl.BlockSpec((_BLK, D), lambda i: (i, 0)),
            pl.BlockSpec((DEG, _BLK, D), lambda i: (0, i, 0)),
            pl.BlockSpec((DEG, _BLK, D), lambda i: (0, i, 0)),
            _full((2 * D, 4 * D)), _full((1, 4 * D)),
            _full((D, D)), _full((1, D)), _full((D, D)),
            _full((2 * D, 4 * D)), _full((1, 4 * D)),
            _full((D, D)), _full((1, D)), _full((D, D)),
            _full((D, D)),
        ],
        out_specs=pl.BlockSpec((_BLK, D), lambda i: (i, 0)),
        out_shape=jax.ShapeDtypeStruct((n_sl, D), jnp.float32),
    )(x_slice, f0, f1, *weights)


def kernel(x, nbr_0, nbr_1, trans,
           W_ih_0, W_hh_0, b_ih_0, b_hh_0, W_self_0, b_self_0, W_neigh_0,
           W_ih_1, W_hh_1, b_ih_1, b_hh_1, W_self_1, b_self_1, W_neigh_1):
    bf = jnp.bfloat16
    gate_scale = jnp.concatenate(
        [jnp.full((2 * D,), 0.5, jnp.float32),
         jnp.ones((D,), jnp.float32),
         jnp.full((D,), 0.5, jnp.float32)])
    wcat0 = (jnp.concatenate([W_ih_0, W_hh_0], axis=1).T
             * gate_scale).astype(bf)
    wcat1 = (jnp.concatenate([W_ih_1, W_hh_1], axis=1).T
             * gate_scale).astype(bf)
    bc0 = ((b_ih_0 + b_hh_0) * gate_scale).reshape(1, 4 * D)
    bc1 = ((b_ih_1 + b_hh_1) * gate_scale).reshape(1, 4 * D)
    weights = (wcat0, bc0, W_self_0.T.astype(bf), b_self_0.reshape(1, D),
               W_neigh_0.T.astype(bf),
               wcat1, bc1, W_self_1.T.astype(bf), b_self_1.reshape(1, D),
               W_neigh_1.T.astype(bf),
               trans.astype(bf))

    gathers = {}
    pad = jnp.zeros((_SUP,), jnp.int32)
    outs = []
    base = 0
    for n_sl in _SLICES:
        if n_sl not in gathers:
            gathers[n_sl] = _make_sc_gather(n_sl)
        sl = slice(base, base + n_sl)
        base += n_sl
        idx0 = jnp.concatenate([nbr_0[sl].astype(jnp.int32).T.reshape(-1),
                                pad])
        idx1 = jnp.concatenate([nbr_1[sl].astype(jnp.int32).T.reshape(-1),
                                pad])
        f0_flat, f1_flat = gathers[n_sl](x, idx0, idx1)
        f0 = f0_flat.reshape(DEG, n_sl, D)
        f1 = f1_flat.reshape(DEG, n_sl, D)
        outs.append(_tc_call(x[sl], f0, f1, weights))
    return jnp.concatenate(outs, axis=0)


# speedup vs baseline: 1.0593x; 1.0243x over previous
import functools

import jax
import jax.numpy as jnp
from jax import lax
from jax.experimental import pallas as pl
from jax.experimental.pallas import tpu as pltpu
from jax.experimental.pallas import tpu_sc as plsc

N = 10000
DEG = 32
D = 128

_SLICES = (1600, 2000, 2000, 2000, 2400)


_SUP = 256


def _make_sc_gather(n_nodes):
    rows_total = n_nodes * DEG
    nsup_tot = rows_total // _SUP
    info = plsc.get_sparse_core_info()
    nw = info.num_cores * info.num_subcores
    nfull = nsup_tot // nw
    extra = nsup_tot % nw
    maxi = (nfull + (1 if extra else 0)) * _SUP
    nit = (nfull + (1 if extra else 0) + 1) // 2

    mesh = plsc.VectorSubcoreMesh(core_axis_name="c", subcore_axis_name="s")

    @functools.partial(
        pl.kernel,
        mesh=mesh,
        out_type=[jax.ShapeDtypeStruct((rows_total, D), jnp.float32)] * 2,
        scratch_types=[
            pltpu.VMEM((2, maxi), jnp.int32),
            pltpu.VMEM((_SUP, D), jnp.float32),
            pltpu.VMEM((_SUP, D), jnp.float32),
            pltpu.SemaphoreType.DMA,
            pltpu.SemaphoreType.DMA,
            pltpu.SemaphoreType.DMA,
            pltpu.SemaphoreType.DMA,
        ],
    )
    def gather(x_hbm, i0_hbm, i1_hbm, o0_hbm, o1_hbm,
               idx_v, rows0, rows1, gsem0, gsem1, osem0, osem1):
        wid = lax.axis_index("s") * info.num_cores + lax.axis_index("c")
        nsup = nfull + jnp.where(wid < extra, 1, 0)
        base_row = (wid * nfull + jnp.minimum(wid, extra)) * _SUP
        rows = (rows0, rows1)
        gsem = (gsem0, gsem1)
        osem = (osem0, osem1)

        pltpu.sync_copy(i0_hbm.at[pl.ds(base_row, maxi)], idx_v.at[0])
        pltpu.sync_copy(i1_hbm.at[pl.ds(base_row, maxi)], idx_v.at[1])

        def start_gathers(r, i, b):
            for k in range(_SUP // 128):
                pltpu.async_copy(
                    x_hbm.at[idx_v.at[r, pl.ds(i * _SUP + k * 128, 128)]],
                    rows[b].at[pl.ds(k * 128, 128)],
                    gsem[b])

        def wait_gathers(b):
            pltpu.make_async_copy(x_hbm.at[pl.ds(0, _SUP)], rows[b],
                                  gsem[b]).wait()

        def start_out(out_hbm, i, b):
            pltpu.async_copy(rows[b],
                             out_hbm.at[pl.ds(base_row + i * _SUP, _SUP)],
                             osem[b])

        def wait_out(out_hbm, b):
            pltpu.make_async_copy(rows[b], out_hbm.at[pl.ds(0, _SUP)],
                                  osem[b]).wait()

        def one_rel(r, out_hbm):
            def body(j, carry):
                for b in (0, 1):
                    i = 2 * j + b

                    @pl.when(i < nsup)
                    def _(i=i, b=b):
                        @pl.when(i >= 2)
                        def _():
                            wait_out(out_hbm, b)

                        start_gathers(r, i, b)

                        @pl.when(i >= 1)
                        def _():
                            wait_gathers(1 - b)
                            start_out(out_hbm, i - 1, 1 - b)

                return carry

            lax.fori_loop(0, nit, body, 0)
            for b in (0, 1):
                @pl.when((nsup - 1) % 2 == b)
                def _(b=b):
                    wait_gathers(b)
                    start_out(out_hbm, nsup - 1, b)
            wait_out(out_hbm, 0)

            @pl.when(nsup >= 2)
            def _():
                wait_out(out_hbm, 1)

        one_rel(0, o0_hbm)
        one_rel(1, o1_hbm)

    return gather


_BLK = 400
_UNROLL = 8


def _tc_body(x_ref, f0_ref, f1_ref,
             wcat0_ref, bc0_ref, ws0_ref, bs0_ref, wn0_ref,
             wcat1_ref, bc1_ref, ws1_ref, bs1_ref, wn1_ref,
             trans_ref, out_ref):
    xb = x_ref[...]
    w0 = wcat0_ref[...]
    w1 = wcat1_ref[...]
    b0 = bc0_ref[...]
    b1 = bc1_ref[...]

    def upd(g, c):
        i_ = 0.5 * jnp.tanh(g[:, 0:D]) + 0.5
        f_ = 0.5 * jnp.tanh(g[:, D:2 * D]) + 0.5
        gg = jnp.tanh(g[:, 2 * D:3 * D])
        o_ = 0.5 * jnp.tanh(g[:, 3 * D:4 * D]) + 0.5
        c2 = f_ * c + i_ * gg
        return o_ * jnp.tanh(c2), c2

    def step(j, carry):
        for k in range(_UNROLL):
            t = _UNROLL * j + k
            h0, c0, h1, c1 = carry
            xh0 = jnp.concatenate([f0_ref[t].astype(jnp.bfloat16),
                                   h0.astype(jnp.bfloat16)], axis=1)
            xh1 = jnp.concatenate([f1_ref[t].astype(jnp.bfloat16),
                                   h1.astype(jnp.bfloat16)], axis=1)
            g0 = jnp.dot(xh0, w0, preferred_element_type=jnp.float32) + b0
            g1 = jnp.dot(xh1, w1, preferred_element_type=jnp.float32) + b1
            h0n, c0n = upd(g0, c0)
            h1n, c1n = upd(g1, c1)
            carry = (h0n, c0n, h1n, c1n)
        return carry

    z = jnp.zeros((_BLK, D), jnp.float32)
    h0, _, h1, _ = lax.fori_loop(0, DEG // _UNROLL, step, (z, z, z, z))

    xbb = xb.astype(jnp.bfloat16)

    def proj(h, ws_ref, bs_ref, wn_ref):
        r = (jnp.dot(xbb, ws_ref[...], preferred_element_type=jnp.float32)
             + bs_ref[...]
             + jnp.dot(h.astype(jnp.bfloat16), wn_ref[...],
                       preferred_element_type=jnp.float32))
        return jnp.where(r > 0, r, 0.01 * r)

    h0 = proj(h0, ws0_ref, bs0_ref, wn0_ref)
    h1 = proj(h1, ws1_ref, bs1_ref, wn1_ref)

    tr = trans_ref[...]
    a0 = jnp.sum(jnp.dot(h0.astype(jnp.bfloat16), tr,
                         preferred_element_type=jnp.float32) * xb,
                 axis=1, keepdims=True)
    a1 = jnp.sum(jnp.dot(h1.astype(jnp.bfloat16), tr,
                         preferred_element_type=jnp.float32) * xb,
                 axis=1, keepdims=True)
    m = jnp.maximum(a0, a1)
    e0 = jnp.exp(a0 - m)
    e1 = jnp.exp(a1 - m)
    out_ref[...] = (e0 * h0 + e1 * h1) / (e0 + e1)


def _full(shape):
    return pl.BlockSpec(shape, lambda i: tuple(0 for _ in shape))


def _tc_call(x_slice, f0, f1, weights):
    n_sl = x_slice.shape[0]
    grid = n_sl // _BLK
    assert n_sl % _BLK == 0
    return pl.pallas_call(
        _tc_body,
        grid=(grid,),
        in_specs=[
            pl.BlockSpec((_BLK, D), lambda i: (i, 0)),
            pl.BlockSpec((DEG, _BLK, D), lambda i: (0, i, 0)),
            pl.BlockSpec((DEG, _BLK, D), lambda i: (0, i, 0)),
            _full((2 * D, 4 * D)), _full((1, 4 * D)),
            _full((D, D)), _full((1, D)), _full((D, D)),
            _full((2 * D, 4 * D)), _full((1, 4 * D)),
            _full((D, D)), _full((1, D)), _full((D, D)),
            _full((D, D)),
        ],
        out_specs=pl.BlockSpec((_BLK, D), lambda i: (i, 0)),
        out_shape=jax.ShapeDtypeStruct((n_sl, D), jnp.float32),
    )(x_slice, f0, f1, *weights)


def kernel(x, nbr_0, nbr_1, trans,
           W_ih_0, W_hh_0, b_ih_0, b_hh_0, W_self_0, b_self_0, W_neigh_0,
           W_ih_1, W_hh_1, b_ih_1, b_hh_1, W_self_1, b_self_1, W_neigh_1):
    bf = jnp.bfloat16
    gate_scale = jnp.concatenate(
        [jnp.full((2 * D,), 0.5, jnp.float32),
         jnp.ones((D,), jnp.float32),
         jnp.full((D,), 0.5, jnp.float32)])
    wcat0 = (jnp.concatenate([W_ih_0, W_hh_0], axis=1).T
             * gate_scale).astype(bf)
    wcat1 = (jnp.concatenate([W_ih_1, W_hh_1], axis=1).T
             * gate_scale).astype(bf)
    bc0 = ((b_ih_0 + b_hh_0) * gate_scale).reshape(1, 4 * D)
    bc1 = ((b_ih_1 + b_hh_1) * gate_scale).reshape(1, 4 * D)
    weights = (wcat0, bc0, W_self_0.T.astype(bf), b_self_0.reshape(1, D),
               W_neigh_0.T.astype(bf),
               wcat1, bc1, W_self_1.T.astype(bf), b_self_1.reshape(1, D),
               W_neigh_1.T.astype(bf),
               trans.astype(bf))

    gathers = {}
    pad = jnp.zeros((_SUP,), jnp.int32)
    outs = []
    base = 0
    for n_sl in _SLICES:
        if n_sl not in gathers:
            gathers[n_sl] = _make_sc_gather(n_sl)
        sl = slice(base, base + n_sl)
        base += n_sl
        idx0 = jnp.concatenate([nbr_0[sl].astype(jnp.int32).T.reshape(-1),
                                pad])
        idx1 = jnp.concatenate([nbr_1[sl].astype(jnp.int32).T.reshape(-1),
                                pad])
        f0_flat, f1_flat = gathers[n_sl](x, idx0, idx1)
        f0 = f0_flat.reshape(DEG, n_sl, D)
        f1 = f1_flat.reshape(DEG, n_sl, D)
        outs.append(_tc_call(x[sl], f0, f1, weights))
    return jnp.concatenate(outs, axis=0)


# speedup vs baseline: 1.0707x; 1.0108x over previous
import functools

import jax
import jax.numpy as jnp
from jax import lax
from jax.experimental import pallas as pl
from jax.experimental.pallas import tpu as pltpu
from jax.experimental.pallas import tpu_sc as plsc

N = 10000
DEG = 32
D = 128

_SLICES = (2000, 2000, 2000, 2000, 2000)


_SUP = 256


def _make_sc_gather(n_nodes):
    rows_total = n_nodes * DEG
    nsup_tot = rows_total // _SUP
    info = plsc.get_sparse_core_info()
    nw = info.num_cores * info.num_subcores
    nfull = nsup_tot // nw
    extra = nsup_tot % nw
    maxi = (nfull + (1 if extra else 0)) * _SUP
    nit = (nfull + (1 if extra else 0) + 1) // 2

    mesh = plsc.VectorSubcoreMesh(core_axis_name="c", subcore_axis_name="s")

    @functools.partial(
        pl.kernel,
        mesh=mesh,
        out_type=[jax.ShapeDtypeStruct((rows_total, D), jnp.float32)] * 2,
        scratch_types=[
            pltpu.VMEM((2, maxi), jnp.int32),
            pltpu.VMEM((_SUP, D), jnp.float32),
            pltpu.VMEM((_SUP, D), jnp.float32),
            pltpu.SemaphoreType.DMA,
            pltpu.SemaphoreType.DMA,
            pltpu.SemaphoreType.DMA,
            pltpu.SemaphoreType.DMA,
        ],
    )
    def gather(x_hbm, i0_hbm, i1_hbm, o0_hbm, o1_hbm,
               idx_v, rows0, rows1, gsem0, gsem1, osem0, osem1):
        wid = lax.axis_index("s") * info.num_cores + lax.axis_index("c")
        nsup = nfull + jnp.where(wid < extra, 1, 0)
        base_row = (wid * nfull + jnp.minimum(wid, extra)) * _SUP
        rows = (rows0, rows1)
        gsem = (gsem0, gsem1)
        osem = (osem0, osem1)

        pltpu.sync_copy(i0_hbm.at[pl.ds(base_row, maxi)], idx_v.at[0])
        pltpu.sync_copy(i1_hbm.at[pl.ds(base_row, maxi)], idx_v.at[1])

        def start_gathers(r, i, b):
            for k in range(_SUP // 128):
                pltpu.async_copy(
                    x_hbm.at[idx_v.at[r, pl.ds(i * _SUP + k * 128, 128)]],
                    rows[b].at[pl.ds(k * 128, 128)],
                    gsem[b])

        def wait_gathers(b):
            pltpu.make_async_copy(x_hbm.at[pl.ds(0, _SUP)], rows[b],
                                  gsem[b]).wait()

        def start_out(out_hbm, i, b):
            pltpu.async_copy(rows[b],
                             out_hbm.at[pl.ds(base_row + i * _SUP, _SUP)],
                             osem[b])

        def wait_out(out_hbm, b):
            pltpu.make_async_copy(rows[b], out_hbm.at[pl.ds(0, _SUP)],
                                  osem[b]).wait()

        def one_rel(r, out_hbm):
            def body(j, carry):
                for b in (0, 1):
                    i = 2 * j + b

                    @pl.when(i < nsup)
                    def _(i=i, b=b):
                        @pl.when(i >= 2)
                        def _():
                            wait_out(out_hbm, b)

                        start_gathers(r, i, b)

                        @pl.when(i >= 1)
                        def _():
                            wait_gathers(1 - b)
                            start_out(out_hbm, i - 1, 1 - b)

                return carry

            lax.fori_loop(0, nit, body, 0)
            for b in (0, 1):
                @pl.when((nsup - 1) % 2 == b)
                def _(b=b):
                    wait_gathers(b)
                    start_out(out_hbm, nsup - 1, b)
            wait_out(out_hbm, 0)

            @pl.when(nsup >= 2)
            def _():
                wait_out(out_hbm, 1)

        one_rel(0, o0_hbm)
        one_rel(1, o1_hbm)

    return gather


_BLK = 400
_UNROLL = 8


def _tc_body(x_ref, f0_ref, f1_ref,
             wcat0_ref, bc0_ref, ws0_ref, bs0_ref, wn0_ref,
             wcat1_ref, bc1_ref, ws1_ref, bs1_ref, wn1_ref,
             trans_ref, out_ref):
    xb = x_ref[...]
    w0 = wcat0_ref[...]
    w1 = wcat1_ref[...]
    b0 = bc0_ref[...]
    b1 = bc1_ref[...]

    def upd(g, c):
        i_ = 0.5 * jnp.tanh(g[:, 0:D]) + 0.5
        f_ = 0.5 * jnp.tanh(g[:, D:2 * D]) + 0.5
        gg = jnp.tanh(g[:, 2 * D:3 * D])
        o_ = 0.5 * jnp.tanh(g[:, 3 * D:4 * D]) + 0.5
        c2 = f_ * c + i_ * gg
        return o_ * jnp.tanh(c2), c2

    def step(j, carry):
        for k in range(_UNROLL):
            t = _UNROLL * j + k
            h0, c0, h1, c1 = carry
            xh0 = jnp.concatenate([f0_ref[t].astype(jnp.bfloat16),
                                   h0.astype(jnp.bfloat16)], axis=1)
            xh1 = jnp.concatenate([f1_ref[t].astype(jnp.bfloat16),
                                   h1.astype(jnp.bfloat16)], axis=1)
            g0 = jnp.dot(xh0, w0, preferred_element_type=jnp.float32) + b0
            g1 = jnp.dot(xh1, w1, preferred_element_type=jnp.float32) + b1
            h0n, c0n = upd(g0, c0)
            h1n, c1n = upd(g1, c1)
            carry = (h0n, c0n, h1n, c1n)
        return carry

    z = jnp.zeros((_BLK, D), jnp.float32)
    h0, _, h1, _ = lax.fori_loop(0, DEG // _UNROLL, step, (z, z, z, z))

    xbb = xb.astype(jnp.bfloat16)

    def proj(h, ws_ref, bs_ref, wn_ref):
        r = (jnp.dot(xbb, ws_ref[...], preferred_element_type=jnp.float32)
             + bs_ref[...]
             + jnp.dot(h.astype(jnp.bfloat16), wn_ref[...],
                       preferred_element_type=jnp.float32))
        return jnp.where(r > 0, r, 0.01 * r)

    h0 = proj(h0, ws0_ref, bs0_ref, wn0_ref)
    h1 = proj(h1, ws1_ref, bs1_ref, wn1_ref)

    tr = trans_ref[...]
    a0 = jnp.sum(jnp.dot(h0.astype(jnp.bfloat16), tr,
                         preferred_element_type=jnp.float32) * xb,
                 axis=1, keepdims=True)
    a1 = jnp.sum(jnp.dot(h1.astype(jnp.bfloat16), tr,
                         preferred_element_type=jnp.float32) * xb,
                 axis=1, keepdims=True)
    m = jnp.maximum(a0, a1)
    e0 = jnp.exp(a0 - m)
    e1 = jnp.exp(a1 - m)
    out_ref[...] = (e0 * h0 + e1 * h1) / (e0 + e1)


def _full(shape):
    return pl.BlockSpec(shape, lambda i: tuple(0 for _ in shape))


def _tc_call(x_slice, f0, f1, weights):
    n_sl = x_slice.shape[0]
    grid = n_sl // _BLK
    assert n_sl % _BLK == 0
    return pl.pallas_call(
        _tc_body,
        grid=(grid,),
        in_specs=[
            pl.BlockSpec((_BLK, D), lambda i: (i, 0)),
            pl.BlockSpec((DEG, _BLK, D), lambda i: (0, i, 0)),
            pl.BlockSpec((DEG, _BLK, D), lambda i: (0, i, 0)),
            _full((2 * D, 4 * D)), _full((1, 4 * D)),
            _full((D, D)), _full((1, D)), _full((D, D)),
            _full((2 * D, 4 * D)), _full((1, 4 * D)),
            _full((D, D)), _full((1, D)), _full((D, D)),
            _full((D, D)),
        ],
        out_specs=pl.BlockSpec((_BLK, D), lambda i: (i, 0)),
        out_shape=jax.ShapeDtypeStruct((n_sl, D), jnp.float32),
    )(x_slice, f0, f1, *weights)


def kernel(x, nbr_0, nbr_1, trans,
           W_ih_0, W_hh_0, b_ih_0, b_hh_0, W_self_0, b_self_0, W_neigh_0,
           W_ih_1, W_hh_1, b_ih_1, b_hh_1, W_self_1, b_self_1, W_neigh_1):
    bf = jnp.bfloat16
    gate_scale = jnp.concatenate(
        [jnp.full((2 * D,), 0.5, jnp.float32),
         jnp.ones((D,), jnp.float32),
         jnp.full((D,), 0.5, jnp.float32)])
    wcat0 = (jnp.concatenate([W_ih_0, W_hh_0], axis=1).T
             * gate_scale).astype(bf)
    wcat1 = (jnp.concatenate([W_ih_1, W_hh_1], axis=1).T
             * gate_scale).astype(bf)
    bc0 = ((b_ih_0 + b_hh_0) * gate_scale).reshape(1, 4 * D)
    bc1 = ((b_ih_1 + b_hh_1) * gate_scale).reshape(1, 4 * D)
    weights = (wcat0, bc0, W_self_0.T.astype(bf), b_self_0.reshape(1, D),
               W_neigh_0.T.astype(bf),
               wcat1, bc1, W_self_1.T.astype(bf), b_self_1.reshape(1, D),
               W_neigh_1.T.astype(bf),
               trans.astype(bf))

    gathers = {}
    pad = jnp.zeros((_SUP,), jnp.int32)
    outs = []
    base = 0
    for n_sl in _SLICES:
        if n_sl not in gathers:
            gathers[n_sl] = _make_sc_gather(n_sl)
        sl = slice(base, base + n_sl)
        base += n_sl
        idx0 = jnp.concatenate([nbr_0[sl].astype(jnp.int32).T.reshape(-1),
                                pad])
        idx1 = jnp.concatenate([nbr_1[sl].astype(jnp.int32).T.reshape(-1),
                                pad])
        f0_flat, f1_flat = gathers[n_sl](x, idx0, idx1)
        f0 = f0_flat.reshape(DEG, n_sl, D)
        f1 = f1_flat.reshape(DEG, n_sl, D)
        outs.append(_tc_call(x[sl], f0, f1, weights))
    return jnp.concatenate(outs, axis=0)


# speedup vs baseline: 1.0911x; 1.0190x over previous
import functools

import jax
import jax.numpy as jnp
from jax import lax
from jax.experimental import pallas as pl
from jax.experimental.pallas import tpu as pltpu
from jax.experimental.pallas import tpu_sc as plsc

N = 10000
DEG = 32
D = 128

_SLICES = (2000, 2000, 2000, 2000, 2000)


_SUP = 256


def _make_sc_gather(n_nodes):
    rows_total = n_nodes * DEG
    nsup_tot = rows_total // _SUP
    info = plsc.get_sparse_core_info()
    nw = info.num_cores * info.num_subcores
    nfull = nsup_tot // nw
    extra = nsup_tot % nw
    maxi = (nfull + (1 if extra else 0)) * _SUP
    nit = (nfull + (1 if extra else 0) + 1) // 2

    mesh = plsc.VectorSubcoreMesh(core_axis_name="c", subcore_axis_name="s")

    @functools.partial(
        pl.kernel,
        mesh=mesh,
        out_type=[jax.ShapeDtypeStruct((rows_total, D), jnp.float32)] * 2,
        scratch_types=[
            pltpu.VMEM((2, maxi), jnp.int32),
            pltpu.VMEM((_SUP, D), jnp.float32),
            pltpu.VMEM((_SUP, D), jnp.float32),
            pltpu.SemaphoreType.DMA,
            pltpu.SemaphoreType.DMA,
            pltpu.SemaphoreType.DMA,
            pltpu.SemaphoreType.DMA,
        ],
    )
    def gather(x_hbm, i0_hbm, i1_hbm, o0_hbm, o1_hbm,
               idx_v, rows0, rows1, gsem0, gsem1, osem0, osem1):
        wid = lax.axis_index("s") * info.num_cores + lax.axis_index("c")
        nsup = nfull + jnp.where(wid < extra, 1, 0)
        base_row = (wid * nfull + jnp.minimum(wid, extra)) * _SUP
        rows = (rows0, rows1)
        gsem = (gsem0, gsem1)
        osem = (osem0, osem1)

        pltpu.sync_copy(i0_hbm.at[pl.ds(base_row, maxi)], idx_v.at[0])
        pltpu.sync_copy(i1_hbm.at[pl.ds(base_row, maxi)], idx_v.at[1])

        def start_gathers(r, i, b):
            for k in range(_SUP // 128):
                pltpu.async_copy(
                    x_hbm.at[idx_v.at[r, pl.ds(i * _SUP + k * 128, 128)]],
                    rows[b].at[pl.ds(k * 128, 128)],
                    gsem[b])

        def wait_gathers(b):
            pltpu.make_async_copy(x_hbm.at[pl.ds(0, _SUP)], rows[b],
                                  gsem[b]).wait()

        def start_out(out_hbm, i, b):
            pltpu.async_copy(rows[b],
                             out_hbm.at[pl.ds(base_row + i * _SUP, _SUP)],
                             osem[b])

        def wait_out(out_hbm, b):
            pltpu.make_async_copy(rows[b], out_hbm.at[pl.ds(0, _SUP)],
                                  osem[b]).wait()

        def one_rel(r, out_hbm):
            def body(j, carry):
                for b in (0, 1):
                    i = 2 * j + b

                    @pl.when(i < nsup)
                    def _(i=i, b=b):
                        @pl.when(i >= 2)
                        def _():
                            wait_out(out_hbm, b)

                        start_gathers(r, i, b)

                        @pl.when(i >= 1)
                        def _():
                            wait_gathers(1 - b)
                            start_out(out_hbm, i - 1, 1 - b)

                return carry

            lax.fori_loop(0, nit, body, 0)
            for b in (0, 1):
                @pl.when((nsup - 1) % 2 == b)
                def _(b=b):
                    wait_gathers(b)
                    start_out(out_hbm, nsup - 1, b)
            wait_out(out_hbm, 0)

            @pl.when(nsup >= 2)
            def _():
                wait_out(out_hbm, 1)

        one_rel(0, o0_hbm)
        one_rel(1, o1_hbm)

    return gather


_BLK = 400
_UNROLL = 16


def _tc_body(x_ref, f0_ref, f1_ref,
             wcat0_ref, bc0_ref, ws0_ref, bs0_ref, wn0_ref,
             wcat1_ref, bc1_ref, ws1_ref, bs1_ref, wn1_ref,
             trans_ref, out_ref):
    xb = x_ref[...]
    w0 = wcat0_ref[...]
    w1 = wcat1_ref[...]
    b0 = bc0_ref[...]
    b1 = bc1_ref[...]

    def upd(g, c):
        i_ = 0.5 * jnp.tanh(g[:, 0:D]) + 0.5
        f_ = 0.5 * jnp.tanh(g[:, D:2 * D]) + 0.5
        gg = jnp.tanh(g[:, 2 * D:3 * D])
        o_ = 0.5 * jnp.tanh(g[:, 3 * D:4 * D]) + 0.5
        c2 = f_ * c + i_ * gg
        return o_ * jnp.tanh(c2), c2

    def step(j, carry):
        for k in range(_UNROLL):
            t = _UNROLL * j + k
            h0, c0, h1, c1 = carry
            xh0 = jnp.concatenate([f0_ref[t].astype(jnp.bfloat16),
                                   h0.astype(jnp.bfloat16)], axis=1)
            xh1 = jnp.concatenate([f1_ref[t].astype(jnp.bfloat16),
                                   h1.astype(jnp.bfloat16)], axis=1)
            g0 = jnp.dot(xh0, w0, preferred_element_type=jnp.float32) + b0
            g1 = jnp.dot(xh1, w1, preferred_element_type=jnp.float32) + b1
            h0n, c0n = upd(g0, c0)
            h1n, c1n = upd(g1, c1)
            carry = (h0n, c0n, h1n, c1n)
        return carry

    z = jnp.zeros((_BLK, D), jnp.float32)
    h0, _, h1, _ = lax.fori_loop(0, DEG // _UNROLL, step, (z, z, z, z))

    xbb = xb.astype(jnp.bfloat16)

    def proj(h, ws_ref, bs_ref, wn_ref):
        r = (jnp.dot(xbb, ws_ref[...], preferred_element_type=jnp.float32)
             + bs_ref[...]
             + jnp.dot(h.astype(jnp.bfloat16), wn_ref[...],
                       preferred_element_type=jnp.float32))
        return jnp.where(r > 0, r, 0.01 * r)

    h0 = proj(h0, ws0_ref, bs0_ref, wn0_ref)
    h1 = proj(h1, ws1_ref, bs1_ref, wn1_ref)

    tr = trans_ref[...]
    a0 = jnp.sum(jnp.dot(h0.astype(jnp.bfloat16), tr,
                         preferred_element_type=jnp.float32) * xb,
                 axis=1, keepdims=True)
    a1 = jnp.sum(jnp.dot(h1.astype(jnp.bfloat16), tr,
                         preferred_element_type=jnp.float32) * xb,
                 axis=1, keepdims=True)
    m = jnp.maximum(a0, a1)
    e0 = jnp.exp(a0 - m)
    e1 = jnp.exp(a1 - m)
    out_ref[...] = (e0 * h0 + e1 * h1) / (e0 + e1)


def _full(shape):
    return pl.BlockSpec(shape, lambda i: tuple(0 for _ in shape))


def _tc_call(x_slice, f0, f1, weights):
    n_sl = x_slice.shape[0]
    grid = n_sl // _BLK
    assert n_sl % _BLK == 0
    return pl.pallas_call(
        _tc_body,
        grid=(grid,),
        in_specs=[
            pl.BlockSpec((_BLK, D), lambda i: (i, 0)),
            pl.BlockSpec((DEG, _BLK, D), lambda i: (0, i, 0)),
            pl.BlockSpec((DEG, _BLK, D), lambda i: (0, i, 0)),
            _full((2 * D, 4 * D)), _full((1, 4 * D)),
            _full((D, D)), _full((1, D)), _full((D, D)),
            _full((2 * D, 4 * D)), _full((1, 4 * D)),
            _full((D, D)), _full((1, D)), _full((D, D)),
            _full((D, D)),
        ],
        out_specs=pl.BlockSpec((_BLK, D), lambda i: (i, 0)),
        out_shape=jax.ShapeDtypeStruct((n_sl, D), jnp.float32),
    )(x_slice, f0, f1, *weights)


def kernel(x, nbr_0, nbr_1, trans,
           W_ih_0, W_hh_0, b_ih_0, b_hh_0, W_self_0, b_self_0, W_neigh_0,
           W_ih_1, W_hh_1, b_ih_1, b_hh_1, W_self_1, b_self_1, W_neigh_1):
    bf = jnp.bfloat16
    gate_scale = jnp.concatenate(
        [jnp.full((2 * D,), 0.5, jnp.float32),
         jnp.ones((D,), jnp.float32),
         jnp.full((D,), 0.5, jnp.float32)])
    wcat0 = (jnp.concatenate([W_ih_0, W_hh_0], axis=1).T
             * gate_scale).astype(bf)
    wcat1 = (jnp.concatenate([W_ih_1, W_hh_1], axis=1).T
             * gate_scale).astype(bf)
    bc0 = ((b_ih_0 + b_hh_0) * gate_scale).reshape(1, 4 * D)
    bc1 = ((b_ih_1 + b_hh_1) * gate_scale).reshape(1, 4 * D)
    weights = (wcat0, bc0, W_self_0.T.astype(bf), b_self_0.reshape(1, D),
               W_neigh_0.T.astype(bf),
               wcat1, bc1, W_self_1.T.astype(bf), b_self_1.reshape(1, D),
               W_neigh_1.T.astype(bf),
               trans.astype(bf))

    gathers = {}
    pad = jnp.zeros((_SUP,), jnp.int32)
    outs = []
    base = 0
    for n_sl in _SLICES:
        if n_sl not in gathers:
            gathers[n_sl] = _make_sc_gather(n_sl)
        sl = slice(base, base + n_sl)
        base += n_sl
        idx0 = jnp.concatenate([nbr_0[sl].astype(jnp.int32).T.reshape(-1),
                                pad])
        idx1 = jnp.concatenate([nbr_1[sl].astype(jnp.int32).T.reshape(-1),
                                pad])
        f0_flat, f1_flat = gathers[n_sl](x, idx0, idx1)
        f0 = f0_flat.reshape(DEG, n_sl, D)
        f1 = f1_flat.reshape(DEG, n_sl, D)
        outs.append(_tc_call(x[sl], f0, f1, weights))
    return jnp.concatenate(outs, axis=0)
